# Initial kernel scaffold; baseline (speedup 1.0000x reference)
import functools

import jax
import jax.numpy as jnp
from jax import lax
from jax.experimental import pallas as pl
from jax.experimental.pallas import tpu as pltpu
from jax.experimental.pallas import tpu_sc as plsc

N_NODES_ = 100000
D_ = 128
N_EDGES_ = 500000

NC = 2
NS = 16
NW = NC * NS

E_PAD = 512000
E_PER_W = E_PAD // NW
CHUNK = 128
N_CHUNKS = E_PER_W // CHUNK
IDX_ROWS = E_PAD // CHUNK


def _sc_body(z_user, z_item, src_idx, tgt_idx, out,
             idx_s, idx_t, u_rows, v_rows, out_v, sem_u, sem_v):
    wid = lax.axis_index("s") * NC + lax.axis_index("c")

    pltpu.sync_copy(src_idx.at[pl.ds(wid * N_CHUNKS, N_CHUNKS)], idx_s)
    pltpu.sync_copy(tgt_idx.at[pl.ds(wid * N_CHUNKS, N_CHUNKS)], idx_t)

    lane = lax.iota(jnp.int32, 16)

    def chunk_body(j, carry):
        cu = pltpu.async_copy(z_user.at[idx_s.at[j]], u_rows, sem_u)
        cv = pltpu.async_copy(z_item.at[idx_t.at[j]], v_rows, sem_v)
        cu.wait()
        cv.wait()

        def group_body(g, carry2):
            rows = g * 16 + lane

            def d_body(d2, acc):
                for k in range(4):
                    col = jnp.full((16,), d2 * 4 + k, jnp.int32)
                    u = plsc.load_gather(u_rows, [rows, col])
                    v = plsc.load_gather(v_rows, [rows, col])
                    acc = acc + u * v
                return acc

            acc = lax.fori_loop(0, 32, d_body, jnp.zeros((16,), jnp.float32))
            prob = 1.0 / (1.0 + jnp.exp(-acc))
            out_v[pl.ds(j * CHUNK + g * 16, 16)] = prob
            return carry2

        return lax.fori_loop(0, CHUNK // 16, group_body, carry)

    lax.fori_loop(0, N_CHUNKS, chunk_body, jnp.int32(0))

    pltpu.sync_copy(out_v, out.at[pl.ds(wid * E_PER_W, E_PER_W)])


@jax.jit
def _sc_call(z_user, z_item, src_idx, tgt_idx):
    mesh = plsc.VectorSubcoreMesh(core_axis_name="c", subcore_axis_name="s")
    f = functools.partial(
        pl.kernel,
        mesh=mesh,
        out_type=jax.ShapeDtypeStruct((E_PAD,), jnp.float32),
        scratch_types=[
            pltpu.VMEM((N_CHUNKS, CHUNK), jnp.int32),
            pltpu.VMEM((N_CHUNKS, CHUNK), jnp.int32),
            pltpu.VMEM((CHUNK, D_), jnp.float32),
            pltpu.VMEM((CHUNK, D_), jnp.float32),
            pltpu.VMEM((E_PER_W,), jnp.float32),
            pltpu.SemaphoreType.DMA,
            pltpu.SemaphoreType.DMA,
        ],
    )(_sc_body)
    return f(z_user, z_item, src_idx, tgt_idx)


def kernel(z_user, z_item, edge_label_index):
    idx = edge_label_index.astype(jnp.int32)
    pad = E_PAD - N_EDGES_
    src = jnp.concatenate([idx[0], jnp.zeros((pad,), jnp.int32)])
    tgt = jnp.concatenate([idx[1], jnp.zeros((pad,), jnp.int32)])
    src = src.reshape(IDX_ROWS, CHUNK)
    tgt = tgt.reshape(IDX_ROWS, CHUNK)
    out = _sc_call(z_user, z_item, src, tgt)
    return out[:N_EDGES_]


# speedup vs baseline: 1.5708x; 1.5708x over previous
import functools

import jax
import jax.numpy as jnp
from jax import lax
from jax.experimental import pallas as pl
from jax.experimental.pallas import tpu as pltpu
from jax.experimental.pallas import tpu_sc as plsc

N_NODES_ = 100000
D_ = 128
N_EDGES_ = 500000

NC = 2
NS = 16
NW = NC * NS

E_PAD = 512000
E_PER_W = E_PAD // NW
CHUNK = 128
N_CHUNKS = E_PER_W // CHUNK


def _sc_body(z_user, z_item, src_idx, tgt_idx, out,
             idx_s, idx_t, u_rows, v_rows, out_v, sem_u, sem_v):
    wid = lax.axis_index("s") * NC + lax.axis_index("c")

    pltpu.sync_copy(src_idx.at[pl.ds(wid * E_PER_W, E_PER_W)], idx_s)
    pltpu.sync_copy(tgt_idx.at[pl.ds(wid * E_PER_W, E_PER_W)], idx_t)

    lane = lax.iota(jnp.int32, 16)
    br4 = (0, 8, 4, 12, 2, 10, 6, 14, 1, 9, 5, 13, 3, 11, 7, 15)

    def _shuffle(x, idx):
        return x.at[idx].get(mode="promise_in_bounds")

    def chunk_body(j, carry):
        cu = pltpu.async_copy(z_user.at[idx_s.at[pl.ds(j * CHUNK, CHUNK)]],
                              u_rows, sem_u)
        cv = pltpu.async_copy(z_item.at[idx_t.at[pl.ds(j * CHUNK, CHUNK)]],
                              v_rows, sem_v)
        cu.wait()
        cv.wait()

        def group_body(g, carry2):
            base = g * 16
            vecs = []
            for e in range(16):
                r = base + br4[e]
                acc = u_rows[r, pl.ds(0, 16)] * v_rows[r, pl.ds(0, 16)]
                for k in range(1, 8):
                    acc = acc + (u_rows[r, pl.ds(k * 16, 16)]
                                 * v_rows[r, pl.ds(k * 16, 16)])
                vecs.append(acc)
            for span in (8, 4, 2, 1):
                m = (lane & span) == 0
                perm = lane ^ span
                nxt = []
                for i in range(0, len(vecs), 2):
                    a2 = vecs[i] + _shuffle(vecs[i], perm)
                    b2 = vecs[i + 1] + _shuffle(vecs[i + 1], perm)
                    nxt.append(jnp.where(m, a2, b2))
                vecs = nxt
            dot = vecs[0]
            prob = 1.0 / (1.0 + jnp.exp(-dot))
            out_v[pl.ds(j * CHUNK + base, 16)] = prob
            return carry2

        return lax.fori_loop(0, CHUNK // 16, group_body, carry)

    lax.fori_loop(0, N_CHUNKS, chunk_body, jnp.int32(0))

    pltpu.sync_copy(out_v, out.at[pl.ds(wid * E_PER_W, E_PER_W)])


@jax.jit
def _sc_call(z_user, z_item, src_idx, tgt_idx):
    mesh = plsc.VectorSubcoreMesh(core_axis_name="c", subcore_axis_name="s")
    f = functools.partial(
        pl.kernel,
        mesh=mesh,
        out_type=jax.ShapeDtypeStruct((E_PAD,), jnp.float32),
        scratch_types=[
            pltpu.VMEM((E_PER_W,), jnp.int32),
            pltpu.VMEM((E_PER_W,), jnp.int32),
            pltpu.VMEM((CHUNK, D_), jnp.float32),
            pltpu.VMEM((CHUNK, D_), jnp.float32),
            pltpu.VMEM((E_PER_W,), jnp.float32),
            pltpu.SemaphoreType.DMA,
            pltpu.SemaphoreType.DMA,
        ],
    )(_sc_body)
    return f(z_user, z_item, src_idx, tgt_idx)


def kernel(z_user, z_item, edge_label_index):
    idx = edge_label_index.astype(jnp.int32)
    pad = E_PAD - N_EDGES_
    src = jnp.concatenate([idx[0], jnp.zeros((pad,), jnp.int32)])
    tgt = jnp.concatenate([idx[1], jnp.zeros((pad,), jnp.int32)])
    out = _sc_call(z_user, z_item, src, tgt)
    return out[:N_EDGES_]


# speedup vs baseline: 2.6841x; 1.7088x over previous
import functools

import jax
import jax.numpy as jnp
from jax import lax
from jax.experimental import pallas as pl
from jax.experimental.pallas import tpu as pltpu
from jax.experimental.pallas import tpu_sc as plsc

N_EDGES_ = 500000
D_ = 128

NC = 2
NS = 16
NW = NC * NS

CHUNK = 128
E_PAD = 507904
E_PER_W = E_PAD // NW
N_CHUNKS = E_PER_W // CHUNK

_BR4 = (0, 8, 4, 12, 2, 10, 6, 14, 1, 9, 5, 13, 3, 11, 7, 15)


def _sc_body(z_user, z_item, src_idx, tgt_idx, out,
             idx_s, idx_t, u0, v0, u1, v1, out_v,
             sem_u0, sem_v0, sem_u1, sem_v1):
    wid = lax.axis_index("s") * NC + lax.axis_index("c")

    pltpu.sync_copy(src_idx.at[pl.ds(wid * E_PER_W, E_PER_W)], idx_s)
    pltpu.sync_copy(tgt_idx.at[pl.ds(wid * E_PER_W, E_PER_W)], idx_t)

    bufs = ((u0, v0, sem_u0, sem_v0), (u1, v1, sem_u1, sem_v1))
    lane = lax.iota(jnp.int32, 16)

    def issue(j, b):
        u_b, v_b, sem_u, sem_v = bufs[b]
        pltpu.async_copy(z_user.at[idx_s.at[pl.ds(j * CHUNK, CHUNK)]],
                         u_b, sem_u)
        pltpu.async_copy(z_item.at[idx_t.at[pl.ds(j * CHUNK, CHUNK)]],
                         v_b, sem_v)

    def wait(b):
        u_b, v_b, sem_u, sem_v = bufs[b]
        pltpu.make_async_copy(z_user.at[idx_s.at[pl.ds(0, CHUNK)]],
                              u_b, sem_u).wait()
        pltpu.make_async_copy(z_item.at[idx_t.at[pl.ds(0, CHUNK)]],
                              v_b, sem_v).wait()

    def compute(j, b):
        u_b, v_b = bufs[b][0], bufs[b][1]

        def group_body(g, carry2):
            base = g * 16
            vecs = []
            for e in range(16):
                r = base + _BR4[e]
                acc = u_b[r, pl.ds(0, 16)] * v_b[r, pl.ds(0, 16)]
                for k in range(1, 8):
                    acc = acc + (u_b[r, pl.ds(k * 16, 16)]
                                 * v_b[r, pl.ds(k * 16, 16)])
                vecs.append(acc)
            for span in (8, 4, 2, 1):
                m = (lane & span) == 0
                perm = lane ^ span
                nxt = []
                for i in range(0, len(vecs), 2):
                    a2 = vecs[i] + vecs[i].at[perm].get(
                        mode="promise_in_bounds")
                    b2 = vecs[i + 1] + vecs[i + 1].at[perm].get(
                        mode="promise_in_bounds")
                    nxt.append(jnp.where(m, a2, b2))
                vecs = nxt
            prob = 1.0 / (1.0 + jnp.exp(-vecs[0]))
            out_v[pl.ds(j * CHUNK + base, 16)] = prob
            return carry2

        lax.fori_loop(0, CHUNK // 16, group_body, jnp.int32(0))

    issue(0, 0)
    issue(1, 1)

    def ring_body(t, carry):
        for b in range(2):
            j = 2 * t + b
            wait(b)
            compute(j, b)
            issue(j + 2, b)
        return carry

    lax.fori_loop(0, (N_CHUNKS - 2) // 2, ring_body, jnp.int32(0))

    for b in range(2):
        j = N_CHUNKS - 2 + b
        wait(b)
        compute(j, b)

    pltpu.sync_copy(out_v, out.at[pl.ds(wid * E_PER_W, E_PER_W)])


@jax.jit
def _sc_call(z_user, z_item, src_idx, tgt_idx):
    mesh = plsc.VectorSubcoreMesh(core_axis_name="c", subcore_axis_name="s")
    f = functools.partial(
        pl.kernel,
        mesh=mesh,
        out_type=jax.ShapeDtypeStruct((E_PAD,), jnp.float32),
        scratch_types=[
            pltpu.VMEM((E_PER_W,), jnp.int32),
            pltpu.VMEM((E_PER_W,), jnp.int32),
            pltpu.VMEM((CHUNK, D_), jnp.float32),
            pltpu.VMEM((CHUNK, D_), jnp.float32),
            pltpu.VMEM((CHUNK, D_), jnp.float32),
            pltpu.VMEM((CHUNK, D_), jnp.float32),
            pltpu.VMEM((E_PER_W,), jnp.float32),
            pltpu.SemaphoreType.DMA,
            pltpu.SemaphoreType.DMA,
            pltpu.SemaphoreType.DMA,
            pltpu.SemaphoreType.DMA,
        ],
    )(_sc_body)
    return f(z_user, z_item, src_idx, tgt_idx)


def kernel(z_user, z_item, edge_label_index):
    idx = edge_label_index.astype(jnp.int32)
    pad = E_PAD - N_EDGES_
    src = jnp.concatenate([idx[0], jnp.zeros((pad,), jnp.int32)])
    tgt = jnp.concatenate([idx[1], jnp.zeros((pad,), jnp.int32)])
    out = _sc_call(z_user, z_item, src, tgt)
    return out[:N_EDGES_]
